# baseline (speedup 1.0000x reference)
import functools

import jax
import jax.numpy as jnp
from jax import lax
from jax.experimental import pallas as pl
from jax.experimental.pallas import tpu as pltpu
from jax.experimental.pallas import tpu_sc as plsc

VOCAB = 49408
EMBED = 512
CTX = 77
N_COMP = 1000
N_HEAD = 4

CTX_PAD = 80
ROWS_PER_WORKER = 8
N_WORKERS_USED = CTX_PAD // ROWS_PER_WORKER

N_TILES = 32
COMPS_PER_TILE = 32
COMP_PAD = N_TILES * COMPS_PER_TILE
LANES = 16
VECS_PER_ROW = EMBED // LANES


def _make_sc_gather():
    mesh = plsc.VectorSubcoreMesh(core_axis_name="c", subcore_axis_name="s")
    info = plsc.get_sparse_core_info()
    num_cores = info.num_cores

    @functools.partial(
        pl.kernel,
        mesh=mesh,
        out_type=jax.ShapeDtypeStruct((CTX_PAD, EMBED), jnp.float32),
        scratch_types=[
            pltpu.VMEM((ROWS_PER_WORKER,), jnp.int32),
            pltpu.VMEM((ROWS_PER_WORKER, EMBED), jnp.float32),
            pltpu.SemaphoreType.DMA,
        ],
    )
    def gather_kernel(table_hbm, idx_hbm, out_hbm, idx_v, rows_v, sem):
        wid = lax.axis_index("s") * num_cores + lax.axis_index("c")

        @pl.when(wid < N_WORKERS_USED)
        def _():
            base = wid * ROWS_PER_WORKER
            pltpu.sync_copy(idx_hbm.at[pl.ds(base, ROWS_PER_WORKER)], idx_v)
            pltpu.async_copy(table_hbm.at[idx_v], rows_v, sem).wait()
            pltpu.sync_copy(rows_v, out_hbm.at[pl.ds(base, ROWS_PER_WORKER)])

    return gather_kernel


def _prep_body(rows_ref, prompt_ref, pos_ref, comp_ref, base_ref, row5_ref):
    pos = pos_ref[0]
    rows = rows_ref[...][:CTX]
    base = rows + pos
    head = prompt_ref[...] + pos[1:N_HEAD + 1]
    base_ref[...] = jnp.concatenate(
        [base[0:1], head, base[N_HEAD + 1:]], axis=0)
    row5_ref[0:N_COMP, :] = comp_ref[:, 0, :] + pos[N_HEAD + 1, :]


def _prep(rows, prompt, pos, comp, interpret=False):
    return pl.pallas_call(
        _prep_body,
        out_shape=[
            jax.ShapeDtypeStruct((CTX, EMBED), jnp.float32),
            jax.ShapeDtypeStruct((COMP_PAD, EMBED), jnp.float32),
        ],
        interpret=interpret,
    )(rows, prompt, pos, comp)


def _make_sc_fanout():
    mesh = plsc.VectorSubcoreMesh(core_axis_name="c", subcore_axis_name="s")
    info = plsc.get_sparse_core_info()
    num_cores = info.num_cores

    @functools.partial(
        pl.kernel,
        mesh=mesh,
        out_type=jax.ShapeDtypeStruct((N_COMP, CTX, EMBED), jnp.float32),
        scratch_types=[
            pltpu.VMEM((CTX, EMBED), jnp.float32),
            pltpu.VMEM((CTX, EMBED), jnp.float32),
            pltpu.VMEM((COMPS_PER_TILE, EMBED), jnp.float32),
            pltpu.SemaphoreType.DMA,
            pltpu.SemaphoreType.DMA,
        ],
    )
    def fanout(base_hbm, row5s_hbm, out_hbm, tmpl_a, tmpl_b, row5_v,
               sem_a, sem_b):
        wid = lax.axis_index("s") * num_cores + lax.axis_index("c")
        cbase = wid * COMPS_PER_TILE
        pltpu.sync_copy(base_hbm, tmpl_a)
        pltpu.sync_copy(base_hbm, tmpl_b)
        pltpu.sync_copy(row5s_hbm.at[pl.ds(cbase, COMPS_PER_TILE)], row5_v)

        bufs = (tmpl_a, tmpl_b)
        sems = (sem_a, sem_b)
        for j in range(COMPS_PER_TILE):
            c = cbase + j
            buf = bufs[j % 2]
            sem = sems[j % 2]

            if j >= 2:
                @pl.when(c - 2 < N_COMP)
                def _():
                    pltpu.make_async_copy(
                        buf, out_hbm.at[c - 2], sem).wait()

            @pl.when(c < N_COMP)
            def _():
                for k in range(VECS_PER_ROW):
                    buf[N_HEAD + 1, pl.ds(k * LANES, LANES)] = (
                        row5_v[j, pl.ds(k * LANES, LANES)])
                pltpu.make_async_copy(buf, out_hbm.at[c], sem).start()

        for j in range(COMPS_PER_TILE - 2, COMPS_PER_TILE):
            c = cbase + j

            @pl.when(c < N_COMP)
            def _():
                pltpu.make_async_copy(
                    bufs[j % 2], out_hbm.at[c], sems[j % 2]).wait()

    return fanout


def kernel(token_table, prompt_vectors_head, comp_embedding, positional_embedding, token_ids):
    ids = token_ids.astype(jnp.int32)
    ids_pad = jnp.zeros((CTX_PAD,), jnp.int32).at[:CTX].set(ids)
    rows = _make_sc_gather()(token_table, ids_pad)
    base, row5s = _prep(rows, prompt_vectors_head, positional_embedding,
                        comp_embedding)
    return _make_sc_fanout()(base, row5s)


# speedup vs baseline: 1.0951x; 1.0951x over previous
import functools

import jax
import jax.numpy as jnp
from jax import lax
from jax.experimental import pallas as pl
from jax.experimental.pallas import tpu as pltpu
from jax.experimental.pallas import tpu_sc as plsc

VOCAB = 49408
EMBED = 512
CTX = 77
N_COMP = 1000
N_HEAD = 4

CTX_PAD = 80
ROWS_PER_WORKER = 8
N_WORKERS_USED = CTX_PAD // ROWS_PER_WORKER

BC = 40


def _make_sc_gather():
    mesh = plsc.VectorSubcoreMesh(core_axis_name="c", subcore_axis_name="s")
    info = plsc.get_sparse_core_info()
    num_cores = info.num_cores

    @functools.partial(
        pl.kernel,
        mesh=mesh,
        out_type=jax.ShapeDtypeStruct((CTX_PAD, EMBED), jnp.float32),
        scratch_types=[
            pltpu.VMEM((ROWS_PER_WORKER,), jnp.int32),
            pltpu.VMEM((ROWS_PER_WORKER, EMBED), jnp.float32),
            pltpu.SemaphoreType.DMA,
        ],
    )
    def gather_kernel(table_hbm, idx_hbm, out_hbm, idx_v, rows_v, sem):
        wid = lax.axis_index("s") * num_cores + lax.axis_index("c")

        @pl.when(wid < N_WORKERS_USED)
        def _():
            base = wid * ROWS_PER_WORKER
            pltpu.sync_copy(idx_hbm.at[pl.ds(base, ROWS_PER_WORKER)], idx_v)
            pltpu.async_copy(table_hbm.at[idx_v], rows_v, sem).wait()
            pltpu.sync_copy(rows_v, out_hbm.at[pl.ds(base, ROWS_PER_WORKER)])

    return gather_kernel


NBUF = 8
BC2 = 8
NSTEPS = N_COMP // BC2


def _tc_body(rows_ref, prompt_ref, pos_ref, comp_ref, out_ref, buf, sem):
    i = pl.program_id(0)
    slot = lax.rem(i, NBUF)

    @pl.when(i == 0)
    def _():
        pos = pos_ref[0]
        rows = rows_ref[...][:CTX]
        base = rows + pos
        head = prompt_ref[...] + pos[1:N_HEAD + 1]
        base = jnp.concatenate([base[0:1], head, base[N_HEAD + 1:]], axis=0)
        buf[...] = jnp.broadcast_to(base[None, None], (NBUF, BC2, CTX, EMBED))

    @pl.when(i >= NBUF)
    def _():
        pltpu.make_async_copy(
            buf.at[slot],
            out_ref.at[pl.ds((i - NBUF) * BC2, BC2)],
            sem.at[slot],
        ).wait()

    comp_rows = comp_ref[pl.ds(i * BC2, BC2), 0, :]
    buf[slot, :, N_HEAD + 1:N_HEAD + 2, :] = (
        comp_rows + pos_ref[0, N_HEAD + 1, :])[:, None, :]

    pltpu.make_async_copy(
        buf.at[slot],
        out_ref.at[pl.ds(i * BC2, BC2)],
        sem.at[slot],
    ).start()

    @pl.when(i == NSTEPS - 1)
    def _():
        for s in range(NSTEPS - NBUF, NSTEPS):
            pltpu.make_async_copy(
                buf.at[s % NBUF],
                out_ref.at[pl.ds(s * BC2, BC2)],
                sem.at[s % NBUF],
            ).wait()


def _broadcast(rows, prompt, comp, pos, interpret=False):
    return pl.pallas_call(
        _tc_body,
        grid=(NSTEPS,),
        in_specs=[
            pl.BlockSpec((CTX_PAD, EMBED), lambda i: (0, 0)),
            pl.BlockSpec((N_HEAD, EMBED), lambda i: (0, 0)),
            pl.BlockSpec((1, CTX, EMBED), lambda i: (0, 0, 0)),
            pl.BlockSpec((N_COMP, 1, EMBED), lambda i: (0, 0, 0)),
        ],
        out_specs=pl.BlockSpec(memory_space=pl.ANY),
        out_shape=jax.ShapeDtypeStruct((N_COMP, CTX, EMBED), jnp.float32),
        scratch_shapes=[
            pltpu.VMEM((NBUF, BC2, CTX, EMBED), jnp.float32),
            pltpu.SemaphoreType.DMA((NBUF,)),
        ],
        interpret=interpret,
    )(rows, prompt, pos, comp)


def kernel(token_table, prompt_vectors_head, comp_embedding, positional_embedding, token_ids):
    ids = token_ids.astype(jnp.int32)
    ids_pad = jnp.zeros((CTX_PAD,), jnp.int32).at[:CTX].set(ids)
    rows = _make_sc_gather()(token_table, ids_pad)
    return _broadcast(rows, prompt_vectors_head, comp_embedding,
                      positional_embedding)


# speedup vs baseline: 1.1033x; 1.0075x over previous
import functools

import jax
import jax.numpy as jnp
from jax import lax
from jax.experimental import pallas as pl
from jax.experimental.pallas import tpu as pltpu
from jax.experimental.pallas import tpu_sc as plsc

VOCAB = 49408
EMBED = 512
CTX = 77
N_COMP = 1000
N_HEAD = 4

CTX_PAD = 80
ROWS_PER_WORKER = 8
N_WORKERS_USED = CTX_PAD // ROWS_PER_WORKER

BC = 40


def _make_sc_gather():
    mesh = plsc.VectorSubcoreMesh(core_axis_name="c", subcore_axis_name="s")
    info = plsc.get_sparse_core_info()
    num_cores = info.num_cores

    @functools.partial(
        pl.kernel,
        mesh=mesh,
        out_type=jax.ShapeDtypeStruct((CTX_PAD, EMBED), jnp.float32),
        scratch_types=[
            pltpu.VMEM((ROWS_PER_WORKER,), jnp.int32),
            pltpu.VMEM((ROWS_PER_WORKER, EMBED), jnp.float32),
            pltpu.SemaphoreType.DMA,
        ],
    )
    def gather_kernel(table_hbm, idx_hbm, out_hbm, idx_v, rows_v, sem):
        wid = lax.axis_index("s") * num_cores + lax.axis_index("c")

        @pl.when(wid < N_WORKERS_USED)
        def _():
            base = wid * ROWS_PER_WORKER
            pltpu.sync_copy(idx_hbm.at[pl.ds(base, ROWS_PER_WORKER)], idx_v)
            pltpu.async_copy(table_hbm.at[idx_v], rows_v, sem).wait()
            pltpu.sync_copy(rows_v, out_hbm.at[pl.ds(base, ROWS_PER_WORKER)])

    return gather_kernel


NBUF = 8
BC2 = 8
NSTEPS = N_COMP // BC2


def _tc_body(rows_ref, prompt_ref, pos_ref, comp_ref, out_ref,
             buf, base_scratch, sem):
    i = pl.program_id(0)
    slot = lax.rem(i, NBUF)

    @pl.when(i == 0)
    def _():
        pos = pos_ref[0]
        rows = rows_ref[...][:CTX]
        base = rows + pos
        head = prompt_ref[...] + pos[1:N_HEAD + 1]
        base_scratch[...] = jnp.concatenate(
            [base[0:1], head, base[N_HEAD + 1:]], axis=0)

    @pl.when(i < NBUF)
    def _():
        buf[slot] = jnp.broadcast_to(base_scratch[...], (BC2, CTX, EMBED))

    @pl.when(i >= NBUF)
    def _():
        pltpu.make_async_copy(
            buf.at[slot],
            out_ref.at[pl.ds((i - NBUF) * BC2, BC2)],
            sem.at[slot],
        ).wait()

    comp_rows = comp_ref[pl.ds(i * BC2, BC2), 0, :]
    buf[slot, :, N_HEAD + 1:N_HEAD + 2, :] = (
        comp_rows + pos_ref[0, N_HEAD + 1, :])[:, None, :]

    pltpu.make_async_copy(
        buf.at[slot],
        out_ref.at[pl.ds(i * BC2, BC2)],
        sem.at[slot],
    ).start()

    @pl.when(i == NSTEPS - 1)
    def _():
        for s in range(NSTEPS - NBUF, NSTEPS):
            pltpu.make_async_copy(
                buf.at[s % NBUF],
                out_ref.at[pl.ds(s * BC2, BC2)],
                sem.at[s % NBUF],
            ).wait()


def _broadcast(rows, prompt, comp, pos, interpret=False):
    return pl.pallas_call(
        _tc_body,
        grid=(NSTEPS,),
        in_specs=[
            pl.BlockSpec((CTX_PAD, EMBED), lambda i: (0, 0)),
            pl.BlockSpec((N_HEAD, EMBED), lambda i: (0, 0)),
            pl.BlockSpec((1, CTX, EMBED), lambda i: (0, 0, 0)),
            pl.BlockSpec((N_COMP, 1, EMBED), lambda i: (0, 0, 0)),
        ],
        out_specs=pl.BlockSpec(memory_space=pl.ANY),
        out_shape=jax.ShapeDtypeStruct((N_COMP, CTX, EMBED), jnp.float32),
        scratch_shapes=[
            pltpu.VMEM((NBUF, BC2, CTX, EMBED), jnp.float32),
            pltpu.VMEM((CTX, EMBED), jnp.float32),
            pltpu.SemaphoreType.DMA((NBUF,)),
        ],
        interpret=interpret,
    )(rows, prompt, pos, comp)


def kernel(token_table, prompt_vectors_head, comp_embedding, positional_embedding, token_ids):
    ids = token_ids.astype(jnp.int32)
    ids_pad = jnp.zeros((CTX_PAD,), jnp.int32).at[:CTX].set(ids)
    rows = _make_sc_gather()(token_table, ids_pad)
    return _broadcast(rows, prompt_vectors_head, comp_embedding,
                      positional_embedding)
